# baseline (speedup 1.0000x reference)
import functools

import jax
import jax.numpy as jnp
from jax import lax
from jax.experimental import pallas as pl
from jax.experimental.pallas import tpu as pltpu
from jax.experimental.pallas import tpu_sc as plsc

B, N, DIM = 16, 1024, 384
H, E, K = 2, 64, 8192
M = B * N
BM = 2048
KC = 2048
NKC = K // KC

_PREC = lax.Precision.DEFAULT


def _make_tc_body(gm):
    def _tc_body(x_ref, proj_ref, cbt_ref, ind_ref, loss_ref):
        m = pl.program_id(0)
        h = pl.program_id(1)
        xb = x_ref[...]
        proj = proj_ref[0]
        xp = jnp.dot(xb, proj, preferred_element_type=jnp.float32,
                     precision=_PREC)
        x2 = jnp.sum(xp * xp, axis=1, keepdims=True)

        best = jnp.full((BM, 1), -jnp.inf, dtype=jnp.float32)
        bidxf = jnp.zeros((BM, 1), dtype=jnp.float32)
        iif = lax.broadcasted_iota(jnp.int32, (BM, KC), 1).astype(jnp.float32)
        for kc in range(NKC):
            cbt = cbt_ref[:, kc * KC:(kc + 1) * KC]
            e2h = 0.5 * jnp.sum(cbt * cbt, axis=0, keepdims=True)
            s = jnp.dot(xp, cbt, preferred_element_type=jnp.float32,
                        precision=_PREC) - e2h
            mx = jnp.max(s, axis=1, keepdims=True)
            idxf = jnp.min(jnp.where(s == mx, iif, jnp.float32(K)),
                           axis=1, keepdims=True) + jnp.float32(kc * KC)
            take = mx > best
            best = jnp.where(take, mx, best)
            bidxf = jnp.where(take, idxf, bidxf)

        ind_ref[0, 0, :] = bidxf[:, 0].astype(jnp.int32)

        @pl.when((m == 0) & (h == 0))
        def _init():
            loss_ref[...] = jnp.zeros((1, 1), jnp.float32)

        loss_ref[...] += jnp.reshape(
            jnp.sum(x2 - 2.0 * best), (1, 1)) * (1.0 / (M * H * E))

    return _tc_body


@functools.cache
def _make_tc_argmax(mrows):
    gm = mrows // BM
    return pl.pallas_call(
        _make_tc_body(gm),
        grid=(gm, H),
        in_specs=[
            pl.BlockSpec((BM, DIM), lambda m, h: (m, 0)),
            pl.BlockSpec((1, DIM, E), lambda m, h: (h, 0, 0)),
            pl.BlockSpec((E, K), lambda m, h: (0, 0)),
        ],
        out_specs=[
            pl.BlockSpec((1, 1, BM), lambda m, h: (h * gm + m, 0, 0)),
            pl.BlockSpec((1, 1), lambda m, h: (0, 0)),
        ],
        out_shape=[
            jax.ShapeDtypeStruct((H * gm, 1, BM), jnp.int32),
            jax.ShapeDtypeStruct((1, 1), jnp.float32),
        ],
        compiler_params=pltpu.CompilerParams(
            dimension_semantics=("arbitrary", "arbitrary")),
    )


@functools.cache
def _make_sc_gather(rows):
    info = plsc.get_sparse_core_info()
    nw = info.num_cores * info.num_subcores
    rb = rows // nw

    def _sc_gather_body(table_hbm, idx_hbm, out_hbm, idx_v, rows_v, sem):
        wid = lax.axis_index("s") * info.num_cores + lax.axis_index("c")
        base = wid * rb
        pltpu.sync_copy(idx_hbm.at[pl.ds(base, rb)], idx_v)
        pltpu.async_copy(table_hbm.at[idx_v], rows_v, sem).wait()
        pltpu.sync_copy(rows_v, out_hbm.at[pl.ds(base, rb)])

    return pl.kernel(
        _sc_gather_body,
        out_type=jax.ShapeDtypeStruct((rows, E), jnp.float32),
        mesh=plsc.VectorSubcoreMesh(core_axis_name="c", subcore_axis_name="s"),
        scratch_types=[
            pltpu.VMEM((rb,), jnp.int32),
            pltpu.VMEM((rb, E), jnp.float32),
            pltpu.SemaphoreType.DMA,
        ],
        compiler_params=pltpu.CompilerParams(use_tc_tiling_on_sc=False),
    )


def _half(x2d_half, rand_projs, cbt, codebook):
    mh = x2d_half.shape[0]
    ind3, loss = _make_tc_argmax(mh)(x2d_half, rand_projs, cbt)
    idx_mh = ind3.reshape(H, mh).T
    zrows = _make_sc_gather(mh * H)(codebook, idx_mh.reshape(mh * H))
    return idx_mh, zrows, loss


def kernel(x, rand_projs, codebook):
    x2d = x.reshape(M, DIM)
    cbt = codebook.T
    m2 = M // 2
    idx_a, z_a, loss_a = _half(x2d[:m2], rand_projs, cbt, codebook)
    idx_b, z_b, loss_b = _half(x2d[m2:], rand_projs, cbt, codebook)
    embed_ind = jnp.concatenate([idx_a, idx_b], axis=0).reshape(B, N, H)
    z_q = jnp.concatenate([z_a, z_b], axis=0).reshape(B, N, H * E)
    loss = (loss_a + loss_b)[0, 0]
    return z_q, embed_ind, loss


# speedup vs baseline: 1.2627x; 1.2627x over previous
import jax
import jax.numpy as jnp
from jax import lax
from jax.experimental import pallas as pl
from jax.experimental.pallas import tpu as pltpu
from jax.experimental.pallas import tpu_sc as plsc

B, N, DIM = 16, 1024, 384
H, E, K = 2, 64, 8192
M = B * N
BM = 2048
KC = 1024
NKC = K // KC
GM = M // BM

_PREC = lax.Precision.DEFAULT


def _tc_body(x_ref, proj_ref, cbt_ref, ind_ref, loss_ref):
    m = pl.program_id(0)
    h = pl.program_id(1)
    xb = x_ref[...]
    proj = proj_ref[0]
    xp = jnp.dot(xb, proj, preferred_element_type=jnp.float32,
                 precision=_PREC)
    x2 = jnp.sum(xp * xp, axis=1, keepdims=True)

    best = jnp.full((BM, 1), -jnp.inf, dtype=jnp.float32)
    bidxf = jnp.zeros((BM, 1), dtype=jnp.float32)
    iif = lax.broadcasted_iota(jnp.int32, (BM, KC), 1).astype(jnp.float32)
    for kc in range(NKC):
        cbt = cbt_ref[:, kc * KC:(kc + 1) * KC]
        e2h = 0.5 * jnp.sum(cbt * cbt, axis=0, keepdims=True)
        s = jnp.dot(xp, cbt, preferred_element_type=jnp.float32,
                    precision=_PREC) - e2h
        mx = jnp.max(s, axis=1, keepdims=True)
        idxf = jnp.min(jnp.where(s == mx, iif, jnp.float32(K)),
                       axis=1, keepdims=True) + jnp.float32(kc * KC)
        take = mx > best
        best = jnp.where(take, mx, best)
        bidxf = jnp.where(take, idxf, bidxf)

    ind_ref[0, 0, :] = bidxf[:, 0].astype(jnp.int32)

    @pl.when((m == 0) & (h == 0))
    def _init():
        loss_ref[...] = jnp.zeros((1, 1), jnp.float32)

    loss_ref[...] += jnp.reshape(jnp.sum(x2 - 2.0 * best), (1, 1))

    @pl.when((m == GM - 1) & (h == H - 1))
    def _final():
        loss_ref[...] = loss_ref[...] * (1.0 / (M * H * E))


def _tc_argmax(x2d, projs, cbt):
    return pl.pallas_call(
        _tc_body,
        grid=(GM, H),
        in_specs=[
            pl.BlockSpec((BM, DIM), lambda m, h: (m, 0)),
            pl.BlockSpec((1, DIM, E), lambda m, h: (h, 0, 0)),
            pl.BlockSpec((E, K), lambda m, h: (0, 0)),
        ],
        out_specs=[
            pl.BlockSpec((1, 1, BM), lambda m, h: (h * GM + m, 0, 0)),
            pl.BlockSpec((1, 1), lambda m, h: (0, 0)),
        ],
        out_shape=[
            jax.ShapeDtypeStruct((H * GM, 1, BM), jnp.int32),
            jax.ShapeDtypeStruct((1, 1), jnp.float32),
        ],
        compiler_params=pltpu.CompilerParams(
            dimension_semantics=("arbitrary", "arbitrary")),
    )(x2d, projs, cbt)


import functools


EP = 128


@functools.cache
def _make_sc_gather():
    info = plsc.get_sparse_core_info()
    nw = info.num_cores * info.num_subcores
    rb = (M * H) // nw

    def _sc_gather_body(table_hbm, idx_hbm, out_hbm, idx_v, rows_v, sem):
        wid = lax.axis_index("s") * info.num_cores + lax.axis_index("c")
        base = wid * rb
        pltpu.sync_copy(idx_hbm.at[pl.ds(base, rb)], idx_v)
        pltpu.async_copy(table_hbm.at[idx_v], rows_v, sem).wait()
        pltpu.sync_copy(rows_v, out_hbm.at[pl.ds(base, rb)])

    return pl.kernel(
        _sc_gather_body,
        out_type=jax.ShapeDtypeStruct((M * H, E), jnp.float32),
        mesh=plsc.VectorSubcoreMesh(core_axis_name="c", subcore_axis_name="s"),
        scratch_types=[
            pltpu.VMEM((rb,), jnp.int32),
            pltpu.VMEM((rb, E), jnp.float32),
            pltpu.SemaphoreType.DMA,
        ],
        compiler_params=pltpu.CompilerParams(use_tc_tiling_on_sc=False),
    )


def kernel(x, rand_projs, codebook):
    x2d = x.reshape(M, DIM)
    cbt = codebook.T
    ind3, loss = _tc_argmax(x2d, rand_projs, cbt)
    idx_mh = ind3.reshape(H, M).T
    embed_ind = idx_mh.reshape(B, N, H)
    idx_flat = idx_mh.reshape(M * H)
    zrows = _make_sc_gather()(codebook, idx_flat)
    z_q = zrows.reshape(B, N, H * E)
    return z_q, embed_ind, loss[0, 0]
